# speedup vs baseline: 73.4136x; 73.4136x over previous
import jax
import jax.numpy as jnp
from jax.experimental import pallas as pl

K_NN = 3
M = 3136
M_PAD = 3200
R_BLOCK = 256


def _knn_body(f_ref, mb_ref, out_ref):
    f = f_ref[...]
    mb = mb_ref[...]
    f_sq = jnp.sum(f * f, axis=1, keepdims=True)
    c_sq = jnp.sum(mb * mb, axis=0, keepdims=True)
    dot = jax.lax.dot_general(
        f, mb, (((1,), (0,)), ((), ())),
        preferred_element_type=jnp.float32)
    dist = (f_sq + c_sq) - 2.0 * dot
    col = jax.lax.broadcasted_iota(jnp.int32, dist.shape, 1)
    inf = jnp.float32(jnp.inf)
    dist = jnp.where(col < M, dist, inf)

    m1 = jnp.min(dist, axis=1, keepdims=True)
    i1 = jnp.min(jnp.where(dist == m1, col, M_PAD), axis=1, keepdims=True)
    d2 = jnp.where(col == i1, inf, dist)
    m2 = jnp.min(d2, axis=1, keepdims=True)
    i2 = jnp.min(jnp.where(d2 == m2, col, M_PAD), axis=1, keepdims=True)
    d3 = jnp.where(col == i2, inf, d2)
    m3 = jnp.min(d3, axis=1, keepdims=True)

    out_ref[:, 0:1] = m1
    out_ref[:, 1:2] = m2
    out_ref[:, 2:3] = m3


@jax.jit
def kernel(target_oriented_features, memory_bank):
    B, HW, D = target_oriented_features.shape
    rows = B * HW
    f = target_oriented_features.reshape(rows, D)
    mb = jnp.pad(memory_bank, ((0, 0), (0, M_PAD - M)))
    out = pl.pallas_call(
        _knn_body,
        grid=(rows // R_BLOCK,),
        in_specs=[
            pl.BlockSpec((R_BLOCK, D), lambda i: (i, 0)),
            pl.BlockSpec((D, M_PAD), lambda i: (0, 0)),
        ],
        out_specs=pl.BlockSpec((R_BLOCK, K_NN), lambda i: (i, 0)),
        out_shape=jax.ShapeDtypeStruct((rows, K_NN), jnp.float32),
    )(f, mb)
    return out.reshape(B, HW, K_NN)


# speedup vs baseline: 103.5431x; 1.4104x over previous
import jax
import jax.numpy as jnp
from jax.experimental import pallas as pl

K_NN = 3
M = 3136
M_PAD = 3200
LANES = 128
N_CHUNK = M_PAD // LANES
R_BLOCK = 256


def _knn_body(f_ref, mb_ref, out_ref):
    f = f_ref[...]
    mb = mb_ref[...]
    c_sq = jnp.sum(mb * mb, axis=0, keepdims=True)
    dot = jax.lax.dot_general(
        f, mb, (((1,), (0,)), ((), ())),
        preferred_element_type=jnp.float32)

    r = f.shape[0]
    inf = jnp.float32(jnp.inf)
    lane = jax.lax.broadcasted_iota(jnp.int32, (r, LANES), 1)
    m1 = jnp.full((r, LANES), inf)
    m2 = jnp.full((r, LANES), inf)
    m3 = jnp.full((r, LANES), inf)
    for j in range(N_CHUNK):
        sl = slice(j * LANES, (j + 1) * LANES)
        v = c_sq[:, sl] - 2.0 * dot[:, sl]
        if (j + 1) * LANES > M:
            v = jnp.where(lane < M - j * LANES, v, inf)
        t = jnp.minimum(m1, v)
        v = jnp.maximum(m1, v)
        m1 = t
        t = jnp.minimum(m2, v)
        v = jnp.maximum(m2, v)
        m2 = t
        m3 = jnp.minimum(m3, v)

    cand = jnp.concatenate([m1, m2, m3], axis=1)
    col = jax.lax.broadcasted_iota(jnp.int32, cand.shape, 1)
    w = cand.shape[1]
    a1 = jnp.min(cand, axis=1, keepdims=True)
    i1 = jnp.min(jnp.where(cand == a1, col, w), axis=1, keepdims=True)
    d2 = jnp.where(col == i1, inf, cand)
    a2 = jnp.min(d2, axis=1, keepdims=True)
    i2 = jnp.min(jnp.where(d2 == a2, col, w), axis=1, keepdims=True)
    d3 = jnp.where(col == i2, inf, d2)
    a3 = jnp.min(d3, axis=1, keepdims=True)

    f_sq = jnp.sum(f * f, axis=1, keepdims=True)
    out_ref[:, 0:1] = a1 + f_sq
    out_ref[:, 1:2] = a2 + f_sq
    out_ref[:, 2:3] = a3 + f_sq


@jax.jit
def kernel(target_oriented_features, memory_bank):
    B, HW, D = target_oriented_features.shape
    rows = B * HW
    f = target_oriented_features.reshape(rows, D)
    mb = jnp.pad(memory_bank, ((0, 0), (0, M_PAD - M)))
    out = pl.pallas_call(
        _knn_body,
        grid=(rows // R_BLOCK,),
        in_specs=[
            pl.BlockSpec((R_BLOCK, D), lambda i: (i, 0)),
            pl.BlockSpec((D, M_PAD), lambda i: (0, 0)),
        ],
        out_specs=pl.BlockSpec((R_BLOCK, K_NN), lambda i: (i, 0)),
        out_shape=jax.ShapeDtypeStruct((rows, K_NN), jnp.float32),
    )(f, mb)
    return out.reshape(B, HW, K_NN)


# speedup vs baseline: 113.9675x; 1.1007x over previous
import jax
import jax.numpy as jnp
from jax.experimental import pallas as pl
from jax.experimental.pallas import tpu as pltpu

K_NN = 3
M = 3136
M_PAD = 3200
LANES = 128
N_CHUNK = M_PAD // LANES
D_PAD = 64
R_BLOCK = 256


def _knn_body(f_ref, mb_ref, out_ref, mba_ref):
    r = f_ref.shape[0]
    d = mb_ref.shape[0]

    @pl.when(pl.program_id(0) == 0)
    def _build_augmented_bank():
        mb = mb_ref[...]
        c_sq = jnp.sum(mb * mb, axis=0, keepdims=True)
        zeros = jnp.zeros((D_PAD - d - 1, M_PAD), jnp.float32)
        mba_ref[...] = jnp.concatenate([-2.0 * mb, c_sq, zeros], axis=0)

    f = f_ref[...]
    f_aug = jnp.concatenate(
        [f, jnp.ones((r, 1), jnp.float32),
         jnp.zeros((r, D_PAD - d - 1), jnp.float32)], axis=1)
    dot = jax.lax.dot_general(
        f_aug, mba_ref[...], (((1,), (0,)), ((), ())),
        preferred_element_type=jnp.float32)

    inf = jnp.float32(jnp.inf)
    lane = jax.lax.broadcasted_iota(jnp.int32, (r, LANES), 1)
    m1 = jnp.full((r, LANES), inf)
    m2 = jnp.full((r, LANES), inf)
    m3 = jnp.full((r, LANES), inf)
    for j in range(N_CHUNK):
        v = dot[:, j * LANES:(j + 1) * LANES]
        if (j + 1) * LANES > M:
            v = jnp.where(lane < M - j * LANES, v, inf)
        t = jnp.minimum(m1, v)
        v = jnp.maximum(m1, v)
        m1 = t
        t = jnp.minimum(m2, v)
        v = jnp.maximum(m2, v)
        m2 = t
        m3 = jnp.minimum(m3, v)

    cand = jnp.concatenate([m1, m2, m3], axis=1)
    col = jax.lax.broadcasted_iota(jnp.int32, cand.shape, 1)
    w = cand.shape[1]
    a1 = jnp.min(cand, axis=1, keepdims=True)
    i1 = jnp.min(jnp.where(cand == a1, col, w), axis=1, keepdims=True)
    d2 = jnp.where(col == i1, inf, cand)
    a2 = jnp.min(d2, axis=1, keepdims=True)
    i2 = jnp.min(jnp.where(d2 == a2, col, w), axis=1, keepdims=True)
    d3 = jnp.where(col == i2, inf, d2)
    a3 = jnp.min(d3, axis=1, keepdims=True)

    f_sq = jnp.sum(f * f, axis=1, keepdims=True)
    out_ref[:, 0:1] = a1 + f_sq
    out_ref[:, 1:2] = a2 + f_sq
    out_ref[:, 2:3] = a3 + f_sq


@jax.jit
def kernel(target_oriented_features, memory_bank):
    B, HW, D = target_oriented_features.shape
    rows = B * HW
    f = target_oriented_features.reshape(rows, D)
    mb = jnp.pad(memory_bank, ((0, 0), (0, M_PAD - M)))
    out = pl.pallas_call(
        _knn_body,
        grid=(rows // R_BLOCK,),
        in_specs=[
            pl.BlockSpec((R_BLOCK, D), lambda i: (i, 0)),
            pl.BlockSpec((D, M_PAD), lambda i: (0, 0)),
        ],
        out_specs=pl.BlockSpec((R_BLOCK, K_NN), lambda i: (i, 0)),
        out_shape=jax.ShapeDtypeStruct((rows, K_NN), jnp.float32),
        scratch_shapes=[pltpu.VMEM((D_PAD, M_PAD), jnp.float32)],
    )(f, mb)
    return out.reshape(B, HW, K_NN)


# speedup vs baseline: 134.5724x; 1.1808x over previous
import jax
import jax.numpy as jnp
from jax.experimental import pallas as pl
from jax.experimental.pallas import tpu as pltpu

K_NN = 3
M = 3136
M_PAD = 3200
LANES = 128
N_CHUNK = M_PAD // LANES
D_PAD = 64
R_BLOCK = 512


def _knn_body(f_ref, mb_ref, out_ref, mba_ref):
    r = f_ref.shape[0]
    d = mb_ref.shape[0]

    @pl.when(pl.program_id(0) == 0)
    def _build_augmented_bank():
        mb = mb_ref[...]
        c_sq = jnp.sum(mb * mb, axis=0, keepdims=True)
        zeros = jnp.zeros((D_PAD - d - 1, M_PAD), jnp.float32)
        mba_ref[...] = jnp.concatenate([-2.0 * mb, c_sq, zeros], axis=0)

    f = f_ref[...]
    f_aug = jnp.concatenate(
        [f, jnp.ones((r, 1), jnp.float32),
         jnp.zeros((r, D_PAD - d - 1), jnp.float32)], axis=1)
    dot = jax.lax.dot_general(
        f_aug, mba_ref[...], (((1,), (0,)), ((), ())),
        preferred_element_type=jnp.float32)

    inf = jnp.float32(jnp.inf)
    lane = jax.lax.broadcasted_iota(jnp.int32, (r, LANES), 1)
    m1 = jnp.full((r, LANES), inf)
    m2 = jnp.full((r, LANES), inf)
    m3 = jnp.full((r, LANES), inf)
    for j in range(N_CHUNK):
        v = dot[:, j * LANES:(j + 1) * LANES]
        if (j + 1) * LANES > M:
            v = jnp.where(lane < M - j * LANES, v, inf)
        t = jnp.minimum(m1, v)
        v = jnp.maximum(m1, v)
        m1 = t
        t = jnp.minimum(m2, v)
        v = jnp.maximum(m2, v)
        m2 = t
        m3 = jnp.minimum(m3, v)

    cand = jnp.concatenate([m1, m2, m3], axis=1)
    col = jax.lax.broadcasted_iota(jnp.int32, cand.shape, 1)
    w = cand.shape[1]
    a1 = jnp.min(cand, axis=1, keepdims=True)
    i1 = jnp.min(jnp.where(cand == a1, col, w), axis=1, keepdims=True)
    d2 = jnp.where(col == i1, inf, cand)
    a2 = jnp.min(d2, axis=1, keepdims=True)
    i2 = jnp.min(jnp.where(d2 == a2, col, w), axis=1, keepdims=True)
    d3 = jnp.where(col == i2, inf, d2)
    a3 = jnp.min(d3, axis=1, keepdims=True)

    f_sq = jnp.sum(f * f, axis=1, keepdims=True)
    out_ref[:, 0:1] = a1 + f_sq
    out_ref[:, 1:2] = a2 + f_sq
    out_ref[:, 2:3] = a3 + f_sq


@jax.jit
def kernel(target_oriented_features, memory_bank):
    B, HW, D = target_oriented_features.shape
    rows = B * HW
    f = target_oriented_features.reshape(rows, D)
    mb = jnp.pad(memory_bank, ((0, 0), (0, M_PAD - M)))
    out = pl.pallas_call(
        _knn_body,
        grid=(rows // R_BLOCK,),
        in_specs=[
            pl.BlockSpec((R_BLOCK, D), lambda i: (i, 0)),
            pl.BlockSpec((D, M_PAD), lambda i: (0, 0)),
        ],
        out_specs=pl.BlockSpec((R_BLOCK, K_NN), lambda i: (i, 0)),
        out_shape=jax.ShapeDtypeStruct((rows, K_NN), jnp.float32),
        scratch_shapes=[pltpu.VMEM((D_PAD, M_PAD), jnp.float32)],
    )(f, mb)
    return out.reshape(B, HW, K_NN)


# speedup vs baseline: 141.7770x; 1.0535x over previous
import jax
import jax.numpy as jnp
from jax.experimental import pallas as pl
from jax.experimental.pallas import tpu as pltpu

K_NN = 3
M = 3136
M_PAD = 3200
LANES = 128
N_CHUNK = M_PAD // LANES
D_PAD = 64
R_BLOCK = 896


def _knn_body(f_ref, mb_ref, out_ref, mba_ref):
    r = f_ref.shape[0]
    d = mb_ref.shape[0]

    @pl.when(pl.program_id(0) == 0)
    def _build_augmented_bank():
        mb = mb_ref[...]
        c_sq = jnp.sum(mb * mb, axis=0, keepdims=True)
        zeros = jnp.zeros((D_PAD - d - 1, M_PAD), jnp.float32)
        mba_ref[...] = jnp.concatenate([-2.0 * mb, c_sq, zeros], axis=0)

    f = f_ref[...]
    f_aug = jnp.concatenate(
        [f, jnp.ones((r, 1), jnp.float32),
         jnp.zeros((r, D_PAD - d - 1), jnp.float32)], axis=1)
    dot = jax.lax.dot_general(
        f_aug, mba_ref[...], (((1,), (0,)), ((), ())),
        preferred_element_type=jnp.float32)

    inf = jnp.float32(jnp.inf)
    lane = jax.lax.broadcasted_iota(jnp.int32, (r, LANES), 1)
    m1 = jnp.full((r, LANES), inf)
    m2 = jnp.full((r, LANES), inf)
    m3 = jnp.full((r, LANES), inf)
    for j in range(N_CHUNK):
        v = dot[:, j * LANES:(j + 1) * LANES]
        if (j + 1) * LANES > M:
            v = jnp.where(lane < M - j * LANES, v, inf)
        t = jnp.minimum(m1, v)
        v = jnp.maximum(m1, v)
        m1 = t
        t = jnp.minimum(m2, v)
        v = jnp.maximum(m2, v)
        m2 = t
        m3 = jnp.minimum(m3, v)

    cand = jnp.concatenate([m1, m2, m3], axis=1)
    col = jax.lax.broadcasted_iota(jnp.int32, cand.shape, 1)
    w = cand.shape[1]
    a1 = jnp.min(cand, axis=1, keepdims=True)
    i1 = jnp.min(jnp.where(cand == a1, col, w), axis=1, keepdims=True)
    d2 = jnp.where(col == i1, inf, cand)
    a2 = jnp.min(d2, axis=1, keepdims=True)
    i2 = jnp.min(jnp.where(d2 == a2, col, w), axis=1, keepdims=True)
    d3 = jnp.where(col == i2, inf, d2)
    a3 = jnp.min(d3, axis=1, keepdims=True)

    f_sq = jnp.sum(f * f, axis=1, keepdims=True)
    out_ref[:, 0:1] = a1 + f_sq
    out_ref[:, 1:2] = a2 + f_sq
    out_ref[:, 2:3] = a3 + f_sq


@jax.jit
def kernel(target_oriented_features, memory_bank):
    B, HW, D = target_oriented_features.shape
    rows = B * HW
    f = target_oriented_features.reshape(rows, D)
    mb = jnp.pad(memory_bank, ((0, 0), (0, M_PAD - M)))
    out = pl.pallas_call(
        _knn_body,
        grid=(rows // R_BLOCK,),
        in_specs=[
            pl.BlockSpec((R_BLOCK, D), lambda i: (i, 0)),
            pl.BlockSpec((D, M_PAD), lambda i: (0, 0)),
        ],
        out_specs=pl.BlockSpec((R_BLOCK, K_NN), lambda i: (i, 0)),
        out_shape=jax.ShapeDtypeStruct((rows, K_NN), jnp.float32),
        scratch_shapes=[pltpu.VMEM((D_PAD, M_PAD), jnp.float32)],
    )(f, mb)
    return out.reshape(B, HW, K_NN)
